# baseline (speedup 1.0000x reference)
import functools

import jax
import jax.numpy as jnp
from jax import lax
from jax.experimental import pallas as pl
from jax.experimental.pallas import tpu as pltpu
from jax.experimental.pallas import tpu_sc as plsc

EMBEDDING_DIM = 64
BATCH = 4096
_GROUP = 4


@functools.lru_cache(maxsize=None)
def _build_gather(batch, dim):
    info = plsc.get_sparse_core_info()
    nc, ns, nl = info.num_cores, info.num_subcores, info.num_lanes
    nw = nc * ns
    b_per_w = batch // nw
    assert batch % (8 * nw) == 0 and b_per_w % _GROUP == 0
    n_groups = b_per_w // _GROUP
    n_chunks = dim // nl

    mesh = plsc.VectorSubcoreMesh(core_axis_name="c", subcore_axis_name="s")

    @functools.partial(
        pl.kernel,
        mesh=mesh,
        out_type=jax.ShapeDtypeStruct((dim, batch), jnp.float32),
        scratch_types=[
            pltpu.VMEM((b_per_w,), jnp.int32),
            pltpu.VMEM((3 * _GROUP, dim, 128), jnp.float32),
            pltpu.VMEM((dim, b_per_w), jnp.float32),
            pltpu.SemaphoreType.DMA,
            pltpu.SemaphoreType.DMA,
            pltpu.SemaphoreType.DMA,
        ],
        compiler_params=pltpu.CompilerParams(
            use_tc_tiling_on_sc=True, needs_layout_passes=False
        ),
    )
    def gather_kernel(
        idx_hbm, tab_t_hbm, out_t_hbm, idx_s, slabs_v, cols_v, sem_a, sem_b, sem_c
    ):
        wid = lax.axis_index("s") * nc + lax.axis_index("c")
        base = wid * b_per_w
        pltpu.sync_copy(idx_hbm.at[pl.ds(base, b_per_w)], idx_s)
        lane_iota = lax.iota(jnp.int32, nl)
        sems = (sem_a, sem_b, sem_c)

        def issue_set(s, chunk, p0):
            for j in range(_GROUP):
                i = chunk[p0 + j]
                slab = pl.multiple_of((i >> 7) << 7, 128)
                for c3 in range(dim // 8):
                    pltpu.async_copy(
                        tab_t_hbm.at[pl.ds(c3 * 8, 8), pl.ds(slab, 128)],
                        slabs_v.at[s * _GROUP + j, pl.ds(c3 * 8, 8)],
                        sems[s],
                    )

        def extract_set(s, chunk, p0, r0):
            for j in range(_GROUP):
                pltpu.make_async_copy(
                    tab_t_hbm.at[:, pl.ds(0, 128)], slabs_v.at[0], sems[s]
                ).wait()
            for j in range(_GROUP):
                i = chunk[p0 + j]
                lane = jnp.broadcast_to(i & 127, (nl,))
                col = jnp.broadcast_to(r0 + p0 + j, (nl,))
                buf = jnp.broadcast_to(s * _GROUP + j, (nl,))
                for k in range(n_chunks):
                    rows = lane_iota + (k * nl)
                    v = plsc.load_gather(slabs_v, [buf, rows, lane])
                    plsc.store_scatter(cols_v, [rows, col], v)

        n_sets = nl // _GROUP
        n_ch = b_per_w // nl

        chunk0 = idx_s[pl.ds(0, nl)]
        issue_set(0, chunk0, 0)
        issue_set(1, chunk0, _GROUP)
        issue_set(2, chunk0, 2 * _GROUP)

        def group(g, carry):
            r0 = pl.multiple_of(g * nl, nl)
            chunk = idx_s[pl.ds(r0, nl)]
            gn = jnp.minimum(g + 1, n_ch - 1)
            rn = pl.multiple_of(gn * nl, nl)
            chunk_n = idx_s[pl.ds(rn, nl)]
            not_last = g < n_ch - 1
            for h in range(n_sets):
                extract_set(h % 3, chunk, h * _GROUP, r0)
                if h + 3 < n_sets:
                    issue_set(h % 3, chunk, (h + 3) * _GROUP)
                else:
                    @pl.when(not_last)
                    def _(h=h):
                        issue_set(h % 3, chunk_n, (h + 3 - n_sets) * _GROUP)
            return carry

        lax.fori_loop(0, n_ch, group, 0)
        pltpu.sync_copy(cols_v, out_t_hbm.at[:, pl.ds(base, b_per_w)])

    return gather_kernel


def kernel(input_index, embeds):
    batch = input_index.shape[0]
    dim = embeds.shape[1]
    idx = input_index.reshape(batch).astype(jnp.int32)
    out_t = _build_gather(batch, dim)(idx, embeds.T)
    return out_t.T.reshape(batch, 1, dim)


# speedup vs baseline: 1.0369x; 1.0369x over previous
import functools

import jax
import jax.numpy as jnp
from jax import lax
from jax.experimental import pallas as pl
from jax.experimental.pallas import tpu as pltpu
from jax.experimental.pallas import tpu_sc as plsc

EMBEDDING_DIM = 64
BATCH = 4096
_GROUP = 4


@functools.lru_cache(maxsize=None)
def _build_gather(batch, dim):
    info = plsc.get_sparse_core_info()
    nc, ns, nl = info.num_cores, info.num_subcores, info.num_lanes
    nw = nc * ns
    b_per_w = batch // nw
    assert batch % (8 * nw) == 0 and b_per_w % _GROUP == 0
    n_groups = b_per_w // _GROUP
    n_chunks = dim // nl

    mesh = plsc.VectorSubcoreMesh(core_axis_name="c", subcore_axis_name="s")

    @functools.partial(
        pl.kernel,
        mesh=mesh,
        out_type=jax.ShapeDtypeStruct((dim, batch), jnp.float32),
        scratch_types=[
            pltpu.VMEM((b_per_w,), jnp.int32),
            pltpu.VMEM((3 * _GROUP, dim, 128), jnp.float32),
            pltpu.VMEM((dim, b_per_w), jnp.float32),
            pltpu.SemaphoreType.DMA,
            pltpu.SemaphoreType.DMA,
            pltpu.SemaphoreType.DMA,
        ],
        compiler_params=pltpu.CompilerParams(
            use_tc_tiling_on_sc=True, needs_layout_passes=False
        ),
    )
    def gather_kernel(
        idx_hbm, tab_t_hbm, out_t_hbm, idx_s, slabs_v, cols_v, sem_a, sem_b, sem_c
    ):
        wid = lax.axis_index("s") * nc + lax.axis_index("c")
        base = wid * b_per_w
        pltpu.sync_copy(idx_hbm.at[pl.ds(base, b_per_w)], idx_s)
        lane_iota = lax.iota(jnp.int32, nl)
        sems = (sem_a, sem_b, sem_c)

        def issue_set(s, chunk, p0):
            for j in range(_GROUP):
                i = chunk[p0 + j]
                slab = pl.multiple_of((i >> 7) << 7, 128)
                for c3 in range(dim // 8):
                    pltpu.async_copy(
                        tab_t_hbm.at[pl.ds(c3 * 8, 8), pl.ds(slab, 128)],
                        slabs_v.at[s * _GROUP + j, pl.ds(c3 * 8, 8)],
                        sems[s],
                    )

        def extract_set(s, chunk, p0, r0):
            for j in range(_GROUP):
                pltpu.make_async_copy(
                    tab_t_hbm.at[:, pl.ds(0, 128)], slabs_v.at[0], sems[s]
                ).wait()
            for j in range(_GROUP):
                i = chunk[p0 + j]
                lane = jnp.broadcast_to(i & 127, (nl,))
                col = jnp.broadcast_to(r0 + p0 + j, (nl,))
                buf = jnp.broadcast_to(s * _GROUP + j, (nl,))
                for k in range(n_chunks):
                    rows = lane_iota + (k * nl)
                    v = plsc.load_gather(slabs_v, [buf, rows, lane])
                    plsc.store_scatter(cols_v, [rows, col], v)

        n_sets = nl // _GROUP
        n_ch = b_per_w // nl

        chunk0 = idx_s[pl.ds(0, nl)]
        issue_set(0, chunk0, 0)
        issue_set(1, chunk0, _GROUP)

        def group(g, carry):
            r0 = pl.multiple_of(g * nl, nl)
            chunk = idx_s[pl.ds(r0, nl)]
            gn = jnp.minimum(g + 1, n_ch - 1)
            rn = pl.multiple_of(gn * nl, nl)
            chunk_n = idx_s[pl.ds(rn, nl)]
            not_last = g < n_ch - 1
            for h in range(n_sets):
                extract_set(h % 2, chunk, h * _GROUP, r0)
                if h + 2 < n_sets:
                    issue_set(h % 2, chunk, (h + 2) * _GROUP)
                else:
                    @pl.when(not_last)
                    def _(h=h):
                        issue_set(h % 2, chunk_n, (h + 2 - n_sets) * _GROUP)
            return carry

        lax.fori_loop(0, n_ch, group, 0)
        pltpu.sync_copy(cols_v, out_t_hbm.at[:, pl.ds(base, b_per_w)])

    return gather_kernel


def kernel(input_index, embeds):
    batch = input_index.shape[0]
    dim = embeds.shape[1]
    idx = input_index.reshape(batch).astype(jnp.int32)
    out_t = _build_gather(batch, dim)(idx, embeds.T)
    return out_t.T.reshape(batch, 1, dim)
